# baseline (speedup 1.0000x reference)
import jax
import jax.numpy as jnp
from jax import lax
from jax.experimental import pallas as pl
from jax.experimental.pallas import tpu as pltpu
from jax.experimental.pallas import tpu_sc as plsc

BATCH = 16384
F = 26
W = 32
L = 16
NC, NS = 2, 16
NW = NC * NS
BPW = BATCH // NW
IPW = BPW * F
C = 128
CHUNKS = BPW // C
IPC = C * F
IROWS = IPC // 128
UNROLL = 2
VOCAB = 1000000
TBL = 131072
STRIPS = -(-VOCAB // 1024)
T4ROWS = STRIPS * 128
VPAD = T4ROWS * 8
HIMASK = -65536


def _tc_pack_transpose_body(tT_ref, out_ref):
    for j in range(TBL // 1024):
        parts = []
        for b in range(8):
            sl = pl.ds(j * 1024 + 128 * b, 128)
            lo = tT_ref[0:16, sl].view(jnp.int32)
            hi = tT_ref[16:32, sl].view(jnp.int32)
            parts.append(((lo >> 16) & 0xFFFF) | (hi & HIMASK))
        blk = jnp.concatenate(parts, axis=0)
        out_ref[pl.ds(j * 128, 128), :] = blk.T


def _to_packed_row_major(tT):
    grid = (pl.cdiv(VOCAB, TBL),)
    out128 = pl.pallas_call(
        _tc_pack_transpose_body,
        grid=grid,
        in_specs=[pl.BlockSpec((W, TBL), lambda i: (0, i))],
        out_specs=pl.BlockSpec((TBL // 8, 128), lambda i: (i, 0)),
        out_shape=jax.ShapeDtypeStruct((T4ROWS, 128), jnp.int32),
    )(tT)
    return out128.reshape(VPAD, L)


def _body(xf_hbm, table_hbm, out_hbm, q_v, rows0, rows1, o0, o1,
          gsem0, gsem1, osem0, osem1):
    rows = (rows0, rows1)
    outv = (o0, o1)
    gsem = (gsem0, gsem1)
    osem = (osem0, osem1)

    wid = lax.axis_index("s") * NC + lax.axis_index("c")

    pltpu.sync_copy(xf_hbm.at[pl.ds(wid * IPW, IPW)], q_v)

    @plsc.parallel_loop(0, IPW // L, step=1, unroll=4)
    def _(t):
        sl = pl.ds(t * L, L)
        v = q_v[sl]
        q_v[sl] = (
            ((v >> 10) << 10) | ((v & 127) << 3) | ((v >> 7) & 7)
        )

    def fire_gathers(g):
        b = g % 2
        return [
            pltpu.async_copy(
                table_hbm.at[q_v.at[pl.ds(g * IPC + j * 128, 128)]],
                rows[b].at[pl.ds(j * 128, 128)],
                gsem[b],
            )
            for j in range(IROWS)
        ]

    def acc_chunk(rows_v, out_v):
        ii = lax.iota(jnp.int32, L)
        d0 = ii >> 3
        d1 = ii & 7

        @plsc.parallel_loop(0, C, step=1, unroll=UNROLL)
        def _(i):
            r0 = i * F

            def unpack(k):
                u = rows_v[r0 + k, pl.ds(0, L)]
                return (
                    plsc.bitcast(u << 16, jnp.float32),
                    plsc.bitcast(u & HIMASK, jnp.float32),
                )

            a = [list(unpack(k)) for k in range(4)]
            for k in range(4, F):
                lo, hi = unpack(k)
                a[k % 4][0] = a[k % 4][0] + lo
                a[k % 4][1] = a[k % 4][1] + hi
            di = jnp.full((L,), i, jnp.int32)
            plsc.store_scatter(
                out_v, [d0, d1, di],
                (a[0][0] + a[1][0]) + (a[2][0] + a[3][0]))
            plsc.store_scatter(
                out_v, [d0 + 2, d1, di],
                (a[0][1] + a[1][1]) + (a[2][1] + a[3][1]))

    d_g, d_out = {}, {}
    d_g[0] = fire_gathers(0)
    for g in range(CHUNKS):
        b = g % 2
        if g + 1 < CHUNKS:
            d_g[g + 1] = fire_gathers(g + 1)
        for d in d_g[g]:
            d.wait()
        if g >= 2:
            d_out[g - 2].wait()
        acc_chunk(rows[b], outv[b])
        row0 = wid * BPW + g * C
        d_out[g] = pltpu.async_copy(
            outv[b],
            out_hbm.at[:, row0 // 128, :, pl.ds(row0 % 128, C)],
            osem[b])

    for g in range(max(0, CHUNKS - 2), CHUNKS):
        d_out[g].wait()


@jax.jit
def _embed_sum(xf, table8):
    mesh = plsc.VectorSubcoreMesh(
        core_axis_name="c", subcore_axis_name="s", num_cores=NC, num_subcores=NS
    )
    run = pl.kernel(
        _body,
        out_type=jax.ShapeDtypeStruct((4, BATCH // 128, 8, 128), jnp.float32),
        mesh=mesh,
        scratch_types=[
            pltpu.VMEM((IPW,), jnp.int32),
            pltpu.VMEM((IPC, L), jnp.int32),
            pltpu.VMEM((IPC, L), jnp.int32),
            pltpu.VMEM((4, 8, C), jnp.float32),
            pltpu.VMEM((4, 8, C), jnp.float32),
            pltpu.SemaphoreType.DMA,
            pltpu.SemaphoreType.DMA,
            pltpu.SemaphoreType.DMA,
            pltpu.SemaphoreType.DMA,
        ],
        compiler_params=pltpu.CompilerParams(
            use_tc_tiling_on_sc=False, needs_layout_passes=False),
    )
    return run(xf, table8)


def kernel(x, embeddings):
    xf = x.astype(jnp.int32).reshape(BATCH * F)
    table8 = _to_packed_row_major(embeddings.T)
    out4d = _embed_sum(xf, table8)
    return out4d.transpose(1, 3, 0, 2).reshape(BATCH, W)


# speedup vs baseline: 1.0068x; 1.0068x over previous
import jax
import jax.numpy as jnp
from jax import lax
from jax.experimental import pallas as pl
from jax.experimental.pallas import tpu as pltpu
from jax.experimental.pallas import tpu_sc as plsc

BATCH = 16384
F = 26
W = 32
L = 16
NC, NS = 2, 16
NW = NC * NS
BPW = BATCH // NW
IPW = BPW * F
C = 64
CHUNKS = BPW // C
IPC = C * F
IROWS = IPC // 128
UNROLL = 2
VOCAB = 1000000
TBL = 131072
STRIPS = -(-VOCAB // 1024)
T4ROWS = STRIPS * 128
VPAD = T4ROWS * 8
HIMASK = -65536


def _tc_pack_transpose_body(tT_ref, out_ref):
    for j in range(TBL // 1024):
        parts = []
        for b in range(8):
            sl = pl.ds(j * 1024 + 128 * b, 128)
            lo = tT_ref[0:16, sl].view(jnp.int32)
            hi = tT_ref[16:32, sl].view(jnp.int32)
            parts.append(((lo >> 16) & 0xFFFF) | (hi & HIMASK))
        blk = jnp.concatenate(parts, axis=0)
        out_ref[pl.ds(j * 128, 128), :] = blk.T


def _to_packed_row_major(tT):
    grid = (pl.cdiv(VOCAB, TBL),)
    out128 = pl.pallas_call(
        _tc_pack_transpose_body,
        grid=grid,
        in_specs=[pl.BlockSpec((W, TBL), lambda i: (0, i))],
        out_specs=pl.BlockSpec((TBL // 8, 128), lambda i: (i, 0)),
        out_shape=jax.ShapeDtypeStruct((T4ROWS, 128), jnp.int32),
    )(tT)
    return out128.reshape(VPAD, L)


def _body(xf_hbm, table_hbm, out_hbm, q_v, rows0, rows1, o0, o1,
          gsem0, gsem1, osem0, osem1):
    rows = (rows0, rows1)
    outv = (o0, o1)
    gsem = (gsem0, gsem1)
    osem = (osem0, osem1)

    wid = lax.axis_index("s") * NC + lax.axis_index("c")

    pltpu.sync_copy(xf_hbm.at[pl.ds(wid * IPW, IPW)], q_v)

    @plsc.parallel_loop(0, IPW // L, step=1, unroll=4)
    def _(t):
        sl = pl.ds(t * L, L)
        v = q_v[sl]
        q_v[sl] = (
            ((v >> 10) << 10) | ((v & 127) << 3) | ((v >> 7) & 7)
        )

    def fire_gathers(g):
        b = g % 2
        return [
            pltpu.async_copy(
                table_hbm.at[q_v.at[pl.ds(g * IPC + j * 128, 128)]],
                rows[b].at[pl.ds(j * 128, 128)],
                gsem[b],
            )
            for j in range(IROWS)
        ]

    def acc_chunk(rows_v, out_v):
        ii = lax.iota(jnp.int32, L)
        d0 = ii >> 3
        d1 = ii & 7

        @plsc.parallel_loop(0, C, step=1, unroll=UNROLL)
        def _(i):
            r0 = i * F

            def unpack(k):
                u = rows_v[r0 + k, pl.ds(0, L)]
                return (
                    plsc.bitcast(u << 16, jnp.float32),
                    plsc.bitcast(u & HIMASK, jnp.float32),
                )

            a = [list(unpack(k)) for k in range(4)]
            for k in range(4, F):
                lo, hi = unpack(k)
                a[k % 4][0] = a[k % 4][0] + lo
                a[k % 4][1] = a[k % 4][1] + hi
            di = jnp.full((L,), i, jnp.int32)
            plsc.store_scatter(
                out_v, [d0, d1, di],
                (a[0][0] + a[1][0]) + (a[2][0] + a[3][0]))
            plsc.store_scatter(
                out_v, [d0 + 2, d1, di],
                (a[0][1] + a[1][1]) + (a[2][1] + a[3][1]))

    d_g, d_out = {}, {}
    d_g[0] = fire_gathers(0)
    for g in range(CHUNKS):
        b = g % 2
        if g + 1 < CHUNKS:
            d_g[g + 1] = fire_gathers(g + 1)
        for d in d_g[g]:
            d.wait()
        if g >= 2:
            d_out[g - 2].wait()
        acc_chunk(rows[b], outv[b])
        row0 = wid * BPW + g * C
        d_out[g] = pltpu.async_copy(
            outv[b],
            out_hbm.at[:, row0 // 128, :, pl.ds(row0 % 128, C)],
            osem[b])

    for g in range(max(0, CHUNKS - 2), CHUNKS):
        d_out[g].wait()


@jax.jit
def _embed_sum(xf, table8):
    mesh = plsc.VectorSubcoreMesh(
        core_axis_name="c", subcore_axis_name="s", num_cores=NC, num_subcores=NS
    )
    run = pl.kernel(
        _body,
        out_type=jax.ShapeDtypeStruct((4, BATCH // 128, 8, 128), jnp.float32),
        mesh=mesh,
        scratch_types=[
            pltpu.VMEM((IPW,), jnp.int32),
            pltpu.VMEM((IPC, L), jnp.int32),
            pltpu.VMEM((IPC, L), jnp.int32),
            pltpu.VMEM((4, 8, C), jnp.float32),
            pltpu.VMEM((4, 8, C), jnp.float32),
            pltpu.SemaphoreType.DMA,
            pltpu.SemaphoreType.DMA,
            pltpu.SemaphoreType.DMA,
            pltpu.SemaphoreType.DMA,
        ],
        compiler_params=pltpu.CompilerParams(
            use_tc_tiling_on_sc=False, needs_layout_passes=False),
    )
    return run(xf, table8)


def kernel(x, embeddings):
    xf = x.astype(jnp.int32).reshape(BATCH * F)
    table8 = _to_packed_row_major(embeddings.T)
    out4d = _embed_sum(xf, table8)
    return out4d.transpose(1, 3, 0, 2).reshape(BATCH, W)
